# baseline (speedup 1.0000x reference)
import functools

import jax
import jax.numpy as jnp
from jax import lax
from jax.experimental import pallas as pl
from jax.experimental.pallas import tpu as pltpu
from jax.experimental.pallas import tpu_sc as plsc

VOCAB = 100000
EMBD = 128
BATCH = 1024


_SC_INFO = plsc.get_sparse_core_info()
_NC = _SC_INFO.num_cores
_NS = _SC_INFO.num_subcores
_NW = _NC * _NS
_B_PER_W = BATCH // _NW


def _sc_gather(table, idx):
  mesh = plsc.VectorSubcoreMesh(core_axis_name="c", subcore_axis_name="s")

  @functools.partial(
      pl.kernel,
      mesh=mesh,
      out_type=jax.ShapeDtypeStruct((BATCH, EMBD), jnp.float32),
      scratch_types=[
          pltpu.VMEM((_B_PER_W,), jnp.int32),
          pltpu.VMEM((_B_PER_W, EMBD), jnp.float32),
          pltpu.SemaphoreType.DMA,
      ],
  )
  def gather_kernel(table_hbm, idx_hbm, out_hbm, idx_v, rows_v, sem):
    wid = lax.axis_index("s") * _NC + lax.axis_index("c")
    base = wid * _B_PER_W
    pltpu.sync_copy(idx_hbm.at[pl.ds(base, _B_PER_W)], idx_v)
    pltpu.async_copy(table_hbm.at[idx_v], rows_v, sem).wait()
    pltpu.sync_copy(rows_v, out_hbm.at[pl.ds(base, _B_PER_W)])

  return gather_kernel(table, idx)


_TILE_V = 4352


def _matmul_body(emb_ref, w_ref, b_ref, out_ref):
  acc = lax.dot_general(
      w_ref[...], emb_ref[...],
      dimension_numbers=(((1,), (1,)), ((), ())),
      preferred_element_type=jnp.float32,
  )
  out_ref[...] = acc + b_ref[...].T


def _tc_matmul(emb, W, brow):
  outT = pl.pallas_call(
      _matmul_body,
      grid=(pl.cdiv(VOCAB, _TILE_V),),
      in_specs=[
          pl.BlockSpec((BATCH, EMBD), lambda i: (0, 0)),
          pl.BlockSpec((_TILE_V, EMBD), lambda i: (i, 0)),
          pl.BlockSpec((1, _TILE_V), lambda i: (0, i)),
      ],
      out_specs=pl.BlockSpec((_TILE_V, BATCH), lambda i: (i, 0)),
      out_shape=jax.ShapeDtypeStruct((VOCAB, BATCH), jnp.float32),
      compiler_params=pltpu.CompilerParams(
          dimension_semantics=("arbitrary",),
      ),
  )(emb, W, brow)
  return outT.T


def kernel(x, embd_table, W, b):
  emb = _sc_gather(embd_table, x.astype(jnp.int32))
  return _tc_matmul(emb, W, b.reshape(1, VOCAB))


# speedup vs baseline: 1.0018x; 1.0018x over previous
import functools

import jax
import jax.numpy as jnp
from jax import lax
from jax.experimental import pallas as pl
from jax.experimental.pallas import tpu as pltpu
from jax.experimental.pallas import tpu_sc as plsc

VOCAB = 100000
EMBD = 128
BATCH = 1024


_SC_INFO = plsc.get_sparse_core_info()
_NC = _SC_INFO.num_cores
_NS = _SC_INFO.num_subcores
_NW = _NC * _NS
_B_PER_W = BATCH // _NW


def _sc_gather(table, idx):
  mesh = plsc.VectorSubcoreMesh(core_axis_name="c", subcore_axis_name="s")

  @functools.partial(
      pl.kernel,
      mesh=mesh,
      out_type=jax.ShapeDtypeStruct((BATCH, EMBD), jnp.float32),
      scratch_types=[
          pltpu.VMEM((_B_PER_W,), jnp.int32),
          pltpu.VMEM((_B_PER_W, EMBD), jnp.float32),
          pltpu.SemaphoreType.DMA,
      ],
  )
  def gather_kernel(table_hbm, idx_hbm, out_hbm, idx_v, rows_v, sem):
    wid = lax.axis_index("s") * _NC + lax.axis_index("c")
    base = wid * _B_PER_W
    pltpu.sync_copy(idx_hbm.at[pl.ds(base, _B_PER_W)], idx_v)
    pltpu.async_copy(table_hbm.at[idx_v], rows_v, sem).wait()
    pltpu.sync_copy(rows_v, out_hbm.at[pl.ds(base, _B_PER_W)])

  return gather_kernel(table, idx)


_TILE_V = 4864


def _matmul_body(emb_ref, w_ref, b_ref, out_ref):
  acc = lax.dot_general(
      w_ref[...], emb_ref[...],
      dimension_numbers=(((1,), (1,)), ((), ())),
      preferred_element_type=jnp.float32,
  )
  out_ref[...] = acc + b_ref[...].T


def _tc_matmul(emb, W, brow):
  outT = pl.pallas_call(
      _matmul_body,
      grid=(pl.cdiv(VOCAB, _TILE_V),),
      in_specs=[
          pl.BlockSpec((BATCH, EMBD), lambda i: (0, 0)),
          pl.BlockSpec((_TILE_V, EMBD), lambda i: (i, 0)),
          pl.BlockSpec((1, _TILE_V), lambda i: (0, i)),
      ],
      out_specs=pl.BlockSpec((_TILE_V, BATCH), lambda i: (i, 0)),
      out_shape=jax.ShapeDtypeStruct((VOCAB, BATCH), jnp.float32),
      compiler_params=pltpu.CompilerParams(
          dimension_semantics=("arbitrary",),
      ),
  )(emb, W, brow)
  return outT.T


def kernel(x, embd_table, W, b):
  emb = _sc_gather(embd_table, x.astype(jnp.int32))
  return _tc_matmul(emb, W, b.reshape(1, VOCAB))


# speedup vs baseline: 1.0047x; 1.0029x over previous
import functools

import jax
import jax.numpy as jnp
from jax import lax
from jax.experimental import pallas as pl
from jax.experimental.pallas import tpu as pltpu
from jax.experimental.pallas import tpu_sc as plsc

VOCAB = 100000
EMBD = 128
BATCH = 1024


_SC_INFO = plsc.get_sparse_core_info()
_NC = _SC_INFO.num_cores
_NS = _SC_INFO.num_subcores
_NW = _NC * _NS
_B_PER_W = BATCH // _NW


def _sc_gather(table, idx):
  mesh = plsc.VectorSubcoreMesh(core_axis_name="c", subcore_axis_name="s")

  @functools.partial(
      pl.kernel,
      mesh=mesh,
      out_type=jax.ShapeDtypeStruct((BATCH, EMBD), jnp.float32),
      scratch_types=[
          pltpu.VMEM((_B_PER_W,), jnp.int32),
          pltpu.VMEM((_B_PER_W, EMBD), jnp.float32),
          pltpu.SemaphoreType.DMA,
      ],
  )
  def gather_kernel(table_hbm, idx_hbm, out_hbm, idx_v, rows_v, sem):
    wid = lax.axis_index("s") * _NC + lax.axis_index("c")
    base = wid * _B_PER_W
    pltpu.sync_copy(idx_hbm.at[pl.ds(base, _B_PER_W)], idx_v)
    pltpu.async_copy(table_hbm.at[idx_v], rows_v, sem).wait()
    pltpu.sync_copy(rows_v, out_hbm.at[pl.ds(base, _B_PER_W)])

  return gather_kernel(table, idx)


_TILE_V = 4864


def _matmul_body(emb_ref, w_ref, b_ref, out_ref):
  acc = lax.dot_general(
      w_ref[...], emb_ref[...],
      dimension_numbers=(((1,), (1,)), ((), ())),
      preferred_element_type=jnp.float32,
  )
  out_ref[...] = acc + b_ref[...].T


def _tc_matmul(emb, W, brow):
  outT = pl.pallas_call(
      _matmul_body,
      grid=(pl.cdiv(VOCAB, _TILE_V),),
      in_specs=[
          pl.BlockSpec((BATCH, EMBD), lambda i: (0, 0)),
          pl.BlockSpec((_TILE_V, EMBD), lambda i: (i, 0)),
          pl.BlockSpec((1, _TILE_V), lambda i: (0, i)),
      ],
      out_specs=pl.BlockSpec((_TILE_V, BATCH), lambda i: (i, 0)),
      out_shape=jax.ShapeDtypeStruct((VOCAB, BATCH), jnp.float32),
      compiler_params=pltpu.CompilerParams(
          dimension_semantics=("arbitrary",),
          skip_device_barrier=True,
      ),
  )(emb, W, brow)
  return outT.T


def kernel(x, embd_table, W, b):
  emb = _sc_gather(embd_table, x.astype(jnp.int32))
  return _tc_matmul(emb, W, b.reshape(1, VOCAB))
